# baseline (speedup 1.0000x reference)
import functools

import jax
import jax.numpy as jnp
from jax.experimental import pallas as pl
from jax.experimental.pallas import tpu as pltpu

NB = 512
CD = 64


def _vq_body(x_ref, cb_ref, out_ref, acc_ref, cba_ref):
    first = (pl.program_id(0) == 0) & (pl.program_id(1) == 0)
    tt = x_ref.shape[2]

    @pl.when(first)
    def _():
        cb0 = cb_ref[...]
        cba_ref[:, :CD] = -2.0 * cb0
        cba_ref[:, CD:] = jnp.sum(cb0 * cb0, axis=1, keepdims=True)

    xt = x_ref[0]
    xn2 = jnp.sum(xt * xt, axis=0, keepdims=True)
    inv = 1.0 / jnp.maximum(jnp.sqrt(xn2), 1e-12)
    xf = xt * inv
    xfn2 = xn2 * (inv * inv)
    xfa = jnp.concatenate([xf, jnp.ones((1, tt), jnp.float32)], axis=0)

    score = jax.lax.dot_general(cba_ref[...], xfa, (((1,), (0,)), ((), ())),
                                preferred_element_type=jnp.float32)

    idx = jnp.argmin(score, axis=0)
    onehot = (jax.lax.broadcasted_iota(jnp.int32, score.shape, 0)
              == idx[None, :]).astype(jnp.float32)

    xd = jax.lax.dot_general(cb_ref[...], onehot, (((0,), (0,)), ((), ())),
                             preferred_element_type=jnp.float32)
    out_ref[0] = xd

    mind = jnp.min(score, axis=0, keepdims=True) + xfn2
    ohm = jnp.concatenate([onehot, mind], axis=0)
    part = jax.lax.dot_general(ohm, jnp.ones((tt, 1), jnp.float32),
                               (((1,), (0,)), ((), ())),
                               preferred_element_type=jnp.float32)

    @pl.when(first)
    def _():
        acc_ref[...] = part

    @pl.when(jnp.logical_not(first))
    def _():
        acc_ref[...] = acc_ref[...] + part


@functools.partial(jax.jit, static_argnames=("tt",))
def _vq(x, codebook, tt=512):
    n, w, t = x.shape
    out, acc = pl.pallas_call(
        _vq_body,
        grid=(n, t // tt),
        in_specs=[
            pl.BlockSpec((1, w, tt), lambda i, j: (i, 0, j)),
            pl.BlockSpec((NB, CD), lambda i, j: (0, 0)),
        ],
        out_specs=[
            pl.BlockSpec((1, w, tt), lambda i, j: (i, 0, j)),
            pl.BlockSpec((NB + 1, 1), lambda i, j: (0, 0)),
        ],
        out_shape=[
            jax.ShapeDtypeStruct((n, w, t), jnp.float32),
            jax.ShapeDtypeStruct((NB + 1, 1), jnp.float32),
        ],
        scratch_shapes=[pltpu.VMEM((NB, CD + 1), jnp.float32)],
    )(x, codebook)
    ntok = n * t
    count = acc[:NB, 0]
    prob = count / jnp.sum(count)
    perplexity = jnp.exp(-jnp.sum(prob * jnp.log(prob + 1e-7)))
    commit_loss = acc[NB, 0] / (ntok * w)
    return out, commit_loss, perplexity


def kernel(x, codebook):
    return _vq(x, codebook)


# speedup vs baseline: 1.0935x; 1.0935x over previous
import functools

import jax
import jax.numpy as jnp
from jax.experimental import pallas as pl
from jax.experimental.pallas import tpu as pltpu

NB = 512
CD = 64


def _vq_body(x_ref, cb_ref, out_ref, cnt_ref, loss_ref, cba_ref):
    first = (pl.program_id(0) == 0) & (pl.program_id(1) == 0)
    tt = x_ref.shape[2]

    @pl.when(first)
    def _():
        cb0 = cb_ref[...]
        cba_ref[:, :CD] = -2.0 * cb0
        cba_ref[:, CD:] = jnp.sum(cb0 * cb0, axis=1, keepdims=True)

    xt = x_ref[0]
    xn2 = jnp.sum(xt * xt, axis=0, keepdims=True)
    inv = 1.0 / jnp.maximum(jnp.sqrt(xn2), 1e-12)
    xf = xt * inv
    xfn2 = xn2 * (inv * inv)
    xfa = jnp.concatenate([xf, jnp.ones((1, tt), jnp.float32)], axis=0)

    score = jax.lax.dot_general(cba_ref[...], xfa, (((1,), (0,)), ((), ())),
                                preferred_element_type=jnp.float32)

    idx = jnp.argmin(score, axis=0)
    onehot = (jax.lax.broadcasted_iota(jnp.int32, score.shape, 0)
              == idx[None, :]).astype(jnp.float32)

    xd = jax.lax.dot_general(cb_ref[...], onehot, (((0,), (0,)), ((), ())),
                             preferred_element_type=jnp.float32)
    out_ref[0] = xd

    mind = jnp.min(score, axis=0, keepdims=True) + xfn2
    cnt = jnp.sum(onehot, axis=1, keepdims=True)
    lsum = jnp.sum(mind).reshape(1, 1)

    @pl.when(first)
    def _():
        cnt_ref[...] = cnt
        loss_ref[...] = lsum

    @pl.when(jnp.logical_not(first))
    def _():
        cnt_ref[...] = cnt_ref[...] + cnt
        loss_ref[...] = loss_ref[...] + lsum


@functools.partial(jax.jit, static_argnames=("tt",))
def _vq(x, codebook, tt=512):
    n, w, t = x.shape
    out, cnt, lsum = pl.pallas_call(
        _vq_body,
        grid=(n, t // tt),
        in_specs=[
            pl.BlockSpec((1, w, tt), lambda i, j: (i, 0, j)),
            pl.BlockSpec((NB, CD), lambda i, j: (0, 0)),
        ],
        out_specs=[
            pl.BlockSpec((1, w, tt), lambda i, j: (i, 0, j)),
            pl.BlockSpec((NB, 1), lambda i, j: (0, 0)),
            pl.BlockSpec((1, 1), lambda i, j: (0, 0)),
        ],
        out_shape=[
            jax.ShapeDtypeStruct((n, w, t), jnp.float32),
            jax.ShapeDtypeStruct((NB, 1), jnp.float32),
            jax.ShapeDtypeStruct((1, 1), jnp.float32),
        ],
        scratch_shapes=[pltpu.VMEM((NB, CD + 1), jnp.float32)],
    )(x, codebook)
    ntok = n * t
    count = cnt[:, 0]
    prob = count / jnp.sum(count)
    perplexity = jnp.exp(-jnp.sum(prob * jnp.log(prob + 1e-7)))
    commit_loss = lsum[0, 0] / (ntok * w)
    return out, commit_loss, perplexity


def kernel(x, codebook):
    return _vq(x, codebook)


# speedup vs baseline: 1.6599x; 1.5180x over previous
import functools

import jax
import jax.numpy as jnp
from jax.experimental import pallas as pl
from jax.experimental.pallas import tpu as pltpu

NB = 512
CD = 64


def _vq_body(x_ref, cb_ref, out_ref, cnt_ref, loss_ref, cba_ref):
    first = (pl.program_id(0) == 0) & (pl.program_id(1) == 0)
    tt = x_ref.shape[2]

    @pl.when(first)
    def _():
        cb0 = cb_ref[...]
        cba_ref[:, :CD] = -2.0 * cb0
        cba_ref[:, CD:] = jnp.sum(cb0 * cb0, axis=1, keepdims=True)

    xt = x_ref[0]
    xn2 = jnp.sum(xt * xt, axis=0, keepdims=True)
    inv = 1.0 / jnp.maximum(jnp.sqrt(xn2), 1e-12)
    xf = xt * inv
    xfn2 = xn2 * (inv * inv)
    xfa = jnp.concatenate([xf, jnp.ones((1, tt), jnp.float32)], axis=0)

    score = jax.lax.dot_general(cba_ref[...], xfa, (((1,), (0,)), ((), ())),
                                preferred_element_type=jnp.float32)

    idx = jnp.argmin(score, axis=0)
    onehot = (jax.lax.broadcasted_iota(jnp.int32, score.shape, 0)
              == idx[None, :]).astype(jnp.float32)

    xd = jax.lax.dot_general(cb_ref[...], onehot, (((0,), (0,)), ((), ())),
                             preferred_element_type=jnp.float32)
    out_ref[0] = xd

    mind = jnp.min(score, axis=0, keepdims=True) + xfn2
    cnt = jnp.sum(onehot, axis=1, keepdims=True)
    lsum = jnp.sum(mind).reshape(1, 1)

    @pl.when(first)
    def _():
        cnt_ref[...] = cnt
        loss_ref[...] = lsum

    @pl.when(jnp.logical_not(first))
    def _():
        cnt_ref[...] = cnt_ref[...] + cnt
        loss_ref[...] = loss_ref[...] + lsum


@functools.partial(jax.jit, static_argnames=("tt",))
def _vq(x, codebook, tt=1024):
    n, w, t = x.shape
    out, cnt, lsum = pl.pallas_call(
        _vq_body,
        grid=(n, t // tt),
        in_specs=[
            pl.BlockSpec((1, w, tt), lambda i, j: (i, 0, j)),
            pl.BlockSpec((NB, CD), lambda i, j: (0, 0)),
        ],
        out_specs=[
            pl.BlockSpec((1, w, tt), lambda i, j: (i, 0, j)),
            pl.BlockSpec((NB, 1), lambda i, j: (0, 0)),
            pl.BlockSpec((1, 1), lambda i, j: (0, 0)),
        ],
        out_shape=[
            jax.ShapeDtypeStruct((n, w, t), jnp.float32),
            jax.ShapeDtypeStruct((NB, 1), jnp.float32),
            jax.ShapeDtypeStruct((1, 1), jnp.float32),
        ],
        scratch_shapes=[pltpu.VMEM((NB, CD + 1), jnp.float32)],
    )(x, codebook)
    ntok = n * t
    count = cnt[:, 0]
    prob = count / jnp.sum(count)
    perplexity = jnp.exp(-jnp.sum(prob * jnp.log(prob + 1e-7)))
    commit_loss = lsum[0, 0] / (ntok * w)
    return out, commit_loss, perplexity


def kernel(x, codebook):
    return _vq(x, codebook)


# speedup vs baseline: 2.0324x; 1.2244x over previous
import functools

import jax
import jax.numpy as jnp
from jax.experimental import pallas as pl
from jax.experimental.pallas import tpu as pltpu

NB = 512
CD = 64


def _vq_body(x_ref, cb_ref, out_ref, cnt_ref, loss_ref, cba_ref):
    first = (pl.program_id(0) == 0) & (pl.program_id(1) == 0)
    tt = x_ref.shape[2]

    @pl.when(first)
    def _():
        cb0 = cb_ref[...]
        cba_ref[:, :CD] = -2.0 * cb0
        cba_ref[:, CD:] = jnp.sum(cb0 * cb0, axis=1, keepdims=True)

    xt = x_ref[0]
    xn2 = jnp.sum(xt * xt, axis=0, keepdims=True)
    inv = 1.0 / jnp.maximum(jnp.sqrt(xn2), 1e-12)
    xf = xt * inv
    xfn2 = xn2 * (inv * inv)
    xfa = jnp.concatenate([xf, jnp.ones((1, tt), jnp.float32)], axis=0)

    score = jax.lax.dot_general(cba_ref[...], xfa, (((1,), (0,)), ((), ())),
                                preferred_element_type=jnp.float32)

    idx = jnp.argmin(score, axis=0)
    onehot = (jax.lax.broadcasted_iota(jnp.int32, score.shape, 0)
              == idx[None, :]).astype(jnp.float32)

    xd = jax.lax.dot_general(cb_ref[...], onehot, (((0,), (0,)), ((), ())),
                             preferred_element_type=jnp.float32)
    out_ref[0] = xd

    mind = jnp.min(score, axis=0, keepdims=True) + xfn2
    cnt = jnp.sum(onehot, axis=1, keepdims=True)
    lsum = jnp.sum(mind).reshape(1, 1)

    @pl.when(first)
    def _():
        cnt_ref[...] = cnt
        loss_ref[...] = lsum

    @pl.when(jnp.logical_not(first))
    def _():
        cnt_ref[...] = cnt_ref[...] + cnt
        loss_ref[...] = loss_ref[...] + lsum


@functools.partial(jax.jit, static_argnames=("tt",))
def _vq(x, codebook, tt=2048):
    n, w, t = x.shape
    out, cnt, lsum = pl.pallas_call(
        _vq_body,
        grid=(n, t // tt),
        in_specs=[
            pl.BlockSpec((1, w, tt), lambda i, j: (i, 0, j)),
            pl.BlockSpec((NB, CD), lambda i, j: (0, 0)),
        ],
        out_specs=[
            pl.BlockSpec((1, w, tt), lambda i, j: (i, 0, j)),
            pl.BlockSpec((NB, 1), lambda i, j: (0, 0)),
            pl.BlockSpec((1, 1), lambda i, j: (0, 0)),
        ],
        out_shape=[
            jax.ShapeDtypeStruct((n, w, t), jnp.float32),
            jax.ShapeDtypeStruct((NB, 1), jnp.float32),
            jax.ShapeDtypeStruct((1, 1), jnp.float32),
        ],
        scratch_shapes=[pltpu.VMEM((NB, CD + 1), jnp.float32)],
    )(x, codebook)
    ntok = n * t
    count = cnt[:, 0]
    prob = count / jnp.sum(count)
    perplexity = jnp.exp(-jnp.sum(prob * jnp.log(prob + 1e-7)))
    commit_loss = lsum[0, 0] / (ntok * w)
    return out, commit_loss, perplexity


def kernel(x, codebook):
    return _vq(x, codebook)
